# Initial kernel scaffold; baseline (speedup 1.0000x reference)
import functools

import jax
import jax.numpy as jnp
from jax import lax
from jax.experimental import pallas as pl
from jax.experimental.pallas import tpu as pltpu
from jax.experimental.pallas import tpu_sc as plsc

N = 10000
E = 320000
C = 128

NC = 2
NS = 16
NW = NC * NS
KCH = 128
NCHUNKS = E // KCH
BASE_CH = NCHUNKS // NW
EXTRA = NCHUNKS - BASE_CH * NW
ROWS_PER_TILE = N // NS

_sc_mesh = plsc.VectorSubcoreMesh(core_axis_name="c", subcore_axis_name="s")


@functools.partial(
    pl.kernel,
    out_type=jax.ShapeDtypeStruct((NC, N, C), jnp.float32),
    mesh=_sc_mesh,
    scratch_types=[
        pltpu.VMEM((KCH,), jnp.int32),
        pltpu.VMEM((KCH,), jnp.int32),
        pltpu.SMEM((KCH,), jnp.float32),
        pltpu.VMEM((KCH, C), jnp.float32),
        pltpu.VMEM_SHARED((N, C), jnp.float32),
        pltpu.SemaphoreType.DMA,
    ],
)
def _segsum(h_hbm, edge_hbm, ew_hbm, zeros_hbm, out_hbm,
            src_v, dst_v, ew_s, rows_v, aggr_sh, gsem):
    cid = lax.axis_index("c")
    sid = lax.axis_index("s")
    wid = sid * NC + cid

    pltpu.sync_copy(zeros_hbm.at[pl.ds(sid * ROWS_PER_TILE, ROWS_PER_TILE)],
                    aggr_sh.at[pl.ds(sid * ROWS_PER_TILE, ROWS_PER_TILE)])
    plsc.subcore_barrier()

    nch = BASE_CH + jnp.where(wid < EXTRA, 1, 0)

    def chunk_body(ci, carry):
        base = (wid + ci * NW) * KCH
        pltpu.sync_copy(edge_hbm.at[0, pl.ds(base, KCH)], src_v)
        pltpu.sync_copy(edge_hbm.at[1, pl.ds(base, KCH)], dst_v)
        pltpu.sync_copy(ew_hbm.at[pl.ds(base, KCH)], ew_s)
        pltpu.async_copy(h_hbm.at[src_v], rows_v, gsem).wait()

        def edge_body(e, c2):
            wv = jnp.full((16,), ew_s[e])
            for j in range(8):
                rows_v[e, pl.ds(j * 16, 16)] = rows_v[e, pl.ds(j * 16, 16)] * wv
            return c2

        lax.fori_loop(0, KCH, edge_body, 0)
        pltpu.sync_copy(rows_v, aggr_sh.at[dst_v], add=True)
        return carry

    lax.fori_loop(0, nch, chunk_body, 0)
    plsc.subcore_barrier()
    pltpu.sync_copy(aggr_sh.at[pl.ds(sid * ROWS_PER_TILE, ROWS_PER_TILE)],
                    out_hbm.at[cid, pl.ds(sid * ROWS_PER_TILE, ROWS_PER_TILE)])


BLK = 2000


def _tc_in_body(x_ref, wt_ref, b_ref, o_ref):
    y = jnp.dot(x_ref[...], wt_ref[...], preferred_element_type=jnp.float32)
    o_ref[...] = jnp.maximum(y + b_ref[...], 0.0)


def _tc_comb_body(p_ref, h_ref, wrelt_ref, wroott_ref, b_ref, g_ref, be_ref,
                  o_ref):
    aggr = p_ref[0] + p_ref[1]
    t = (jnp.dot(aggr, wrelt_ref[...], preferred_element_type=jnp.float32)
         + jnp.dot(h_ref[...], wroott_ref[...],
                   preferred_element_type=jnp.float32)
         + b_ref[...])
    m = jnp.mean(t, axis=-1, keepdims=True)
    v = jnp.mean((t - m) * (t - m), axis=-1, keepdims=True)
    t = (t - m) * lax.rsqrt(v + 1e-5) * g_ref[...] + be_ref[...]
    o_ref[...] = jnp.maximum(t, 0.0)


def _tc_out_body(p_ref, x1_ref, h_ref, wrelt_ref, wroott_ref, b_ref, g_ref,
                 be_ref, woutt_ref, bout_ref, x2_ref, out_ref):
    aggr = p_ref[0] + p_ref[1]
    t = (jnp.dot(aggr, wrelt_ref[...], preferred_element_type=jnp.float32)
         + jnp.dot(x1_ref[...], wroott_ref[...],
                   preferred_element_type=jnp.float32)
         + b_ref[...])
    m = jnp.mean(t, axis=-1, keepdims=True)
    v = jnp.mean((t - m) * (t - m), axis=-1, keepdims=True)
    t = (t - m) * lax.rsqrt(v + 1e-5) * g_ref[...] + be_ref[...]
    x2 = jnp.maximum(t + h_ref[...], 0.0)
    x2_ref[...] = x2
    out_ref[...] = (jnp.dot(x2, woutt_ref[...],
                            preferred_element_type=jnp.float32)
                    + bout_ref[...])


def _row_spec(blk):
    return pl.BlockSpec((blk, C), lambda i: (i, 0))


_W_SPEC = pl.BlockSpec((C, C), lambda i: (0, 0))
_V_SPEC = pl.BlockSpec((1, C), lambda i: (0, 0))
_P_SPEC = pl.BlockSpec((NC, BLK, C), lambda i: (0, i, 0))

_tc_in = pl.pallas_call(
    _tc_in_body,
    grid=(N // BLK,),
    in_specs=[_row_spec(BLK), _W_SPEC, _V_SPEC],
    out_specs=_row_spec(BLK),
    out_shape=jax.ShapeDtypeStruct((N, C), jnp.float32),
)

_tc_comb = pl.pallas_call(
    _tc_comb_body,
    grid=(N // BLK,),
    in_specs=[_P_SPEC, _row_spec(BLK), _W_SPEC, _W_SPEC, _V_SPEC, _V_SPEC,
              _V_SPEC],
    out_specs=_row_spec(BLK),
    out_shape=jax.ShapeDtypeStruct((N, C), jnp.float32),
)

_tc_out = pl.pallas_call(
    _tc_out_body,
    grid=(N // BLK,),
    in_specs=[_P_SPEC, _row_spec(BLK), _row_spec(BLK), _W_SPEC, _W_SPEC,
              _V_SPEC, _V_SPEC, _V_SPEC, _W_SPEC, _V_SPEC],
    out_specs=[_row_spec(BLK), _row_spec(BLK)],
    out_shape=[jax.ShapeDtypeStruct((N, C), jnp.float32),
               jax.ShapeDtypeStruct((N, C), jnp.float32)],
)


def kernel(x, edge, edgeweight, W_in, b_in, Wrel0, brel0, Wroot0, g0, be0,
           Wrel1, brel1, Wroot1, g1, be1, W_out, b_out):
    zeros = jnp.zeros((N, C), jnp.float32)
    h = _tc_in(x, W_in.T, b_in.reshape(1, C))
    p0 = _segsum(h, edge, edgeweight, zeros)
    x1 = _tc_comb(p0, h, Wrel0.T, Wroot0.T, brel0.reshape(1, C),
                  g0.reshape(1, C), be0.reshape(1, C))
    p1 = _segsum(x1, edge, edgeweight, zeros)
    x2, out = _tc_out(p1, x1, h, Wrel1.T, Wroot1.T, brel1.reshape(1, C),
                      g1.reshape(1, C), be1.reshape(1, C), W_out.T,
                      b_out.reshape(1, C))
    return (x2, out)


# speedup vs baseline: 4.8143x; 4.8143x over previous
import functools

import jax
import jax.numpy as jnp
from jax import lax
from jax.experimental import pallas as pl
from jax.experimental.pallas import tpu as pltpu
from jax.experimental.pallas import tpu_sc as plsc

N = 10000
E = 320000
C = 128

NC = 2
NS = 16
NW = NC * NS
KCH = 128
NCHUNKS = E // KCH
BASE_CH = NCHUNKS // NW
EXTRA = NCHUNKS - BASE_CH * NW
RPT = 624
TAIL_OFF = RPT * NS
TAIL = N - TAIL_OFF

def _segsum_body(h_hbm, edge_hbm, ew_hbm, zeros_hbm, out_hbm,
                 src_v, dst_v, ew_v, rows_v, aggr_sh, gsem):
    cid = lax.axis_index("c")
    sid = lax.axis_index("s")
    wid = sid * NC + cid

    pltpu.sync_copy(zeros_hbm.at[pl.ds(sid * RPT, RPT)],
                    aggr_sh.at[pl.ds(sid * RPT, RPT)])

    @pl.when(sid == NS - 1)
    def _zero_tail():
        pltpu.sync_copy(zeros_hbm.at[pl.ds(TAIL_OFF, TAIL)],
                        aggr_sh.at[pl.ds(TAIL_OFF, TAIL)])

    plsc.subcore_barrier()

    nch = BASE_CH + jnp.where(wid < EXTRA, 1, 0)

    def chunk_body(ci, carry):
        base = (wid + ci * NW) * KCH
        pltpu.sync_copy(edge_hbm.at[0, pl.ds(base, KCH)], src_v)
        pltpu.sync_copy(edge_hbm.at[1, pl.ds(base, KCH)], dst_v)
        pltpu.sync_copy(ew_hbm.at[pl.ds(base, KCH)], ew_v)
        pltpu.async_copy(h_hbm.at[src_v], rows_v, gsem).wait()

        def grp_body(g, c2):
            w16 = ew_v[pl.ds(g * 16, 16)]
            for e2 in range(16):
                wv = lax.gather(
                    w16, jnp.full((16, 1), e2, jnp.int32),
                    lax.GatherDimensionNumbers(
                        offset_dims=(), collapsed_slice_dims=(0,),
                        start_index_map=(0,)),
                    (1,), mode=lax.GatherScatterMode.PROMISE_IN_BOUNDS)
                e = g * 16 + e2
                for j in range(8):
                    rows_v[e, pl.ds(j * 16, 16)] = (
                        rows_v[e, pl.ds(j * 16, 16)] * wv)
            return c2

        lax.fori_loop(0, KCH // 16, grp_body, 0)
        pltpu.sync_copy(rows_v, aggr_sh.at[dst_v], add=True)
        return carry

    lax.fori_loop(0, nch, chunk_body, 0)
    plsc.subcore_barrier()
    pltpu.sync_copy(aggr_sh.at[pl.ds(sid * RPT, RPT)],
                    out_hbm.at[cid, pl.ds(sid * RPT, RPT)])

    @pl.when(sid == NS - 1)
    def _write_tail():
        pltpu.sync_copy(aggr_sh.at[pl.ds(TAIL_OFF, TAIL)],
                        out_hbm.at[cid, pl.ds(TAIL_OFF, TAIL)])


@functools.cache
def _segsum():
    mesh = plsc.VectorSubcoreMesh(core_axis_name="c", subcore_axis_name="s",
                                  num_cores=NC, num_subcores=NS)
    return pl.kernel(
        _segsum_body,
        out_type=jax.ShapeDtypeStruct((NC, N, C), jnp.float32),
        mesh=mesh,
        scratch_types=[
            pltpu.VMEM((KCH,), jnp.int32),
            pltpu.VMEM((KCH,), jnp.int32),
            pltpu.VMEM((KCH,), jnp.float32),
            pltpu.VMEM((KCH, C), jnp.float32),
            pltpu.VMEM_SHARED((N, C), jnp.float32),
            pltpu.SemaphoreType.DMA,
        ],
    )


BLK = 2000


def _tc_in_body(x_ref, wt_ref, b_ref, o_ref):
    y = jnp.dot(x_ref[...], wt_ref[...], preferred_element_type=jnp.float32)
    o_ref[...] = jnp.maximum(y + b_ref[...], 0.0)


def _tc_comb_body(p_ref, h_ref, wrelt_ref, wroott_ref, b_ref, g_ref, be_ref,
                  o_ref):
    aggr = p_ref[0] + p_ref[1]
    t = (jnp.dot(aggr, wrelt_ref[...], preferred_element_type=jnp.float32)
         + jnp.dot(h_ref[...], wroott_ref[...],
                   preferred_element_type=jnp.float32)
         + b_ref[...])
    m = jnp.mean(t, axis=-1, keepdims=True)
    v = jnp.mean((t - m) * (t - m), axis=-1, keepdims=True)
    t = (t - m) * lax.rsqrt(v + 1e-5) * g_ref[...] + be_ref[...]
    o_ref[...] = jnp.maximum(t, 0.0)


def _tc_out_body(p_ref, x1_ref, h_ref, wrelt_ref, wroott_ref, b_ref, g_ref,
                 be_ref, woutt_ref, bout_ref, x2_ref, out_ref):
    aggr = p_ref[0] + p_ref[1]
    t = (jnp.dot(aggr, wrelt_ref[...], preferred_element_type=jnp.float32)
         + jnp.dot(x1_ref[...], wroott_ref[...],
                   preferred_element_type=jnp.float32)
         + b_ref[...])
    m = jnp.mean(t, axis=-1, keepdims=True)
    v = jnp.mean((t - m) * (t - m), axis=-1, keepdims=True)
    t = (t - m) * lax.rsqrt(v + 1e-5) * g_ref[...] + be_ref[...]
    x2 = jnp.maximum(t + h_ref[...], 0.0)
    x2_ref[...] = x2
    out_ref[...] = (jnp.dot(x2, woutt_ref[...],
                            preferred_element_type=jnp.float32)
                    + bout_ref[...])


def _row_spec(blk):
    return pl.BlockSpec((blk, C), lambda i: (i, 0))


_W_SPEC = pl.BlockSpec((C, C), lambda i: (0, 0))
_V_SPEC = pl.BlockSpec((1, C), lambda i: (0, 0))
_P_SPEC = pl.BlockSpec((NC, BLK, C), lambda i: (0, i, 0))

_tc_in = pl.pallas_call(
    _tc_in_body,
    grid=(N // BLK,),
    in_specs=[_row_spec(BLK), _W_SPEC, _V_SPEC],
    out_specs=_row_spec(BLK),
    out_shape=jax.ShapeDtypeStruct((N, C), jnp.float32),
)

_tc_comb = pl.pallas_call(
    _tc_comb_body,
    grid=(N // BLK,),
    in_specs=[_P_SPEC, _row_spec(BLK), _W_SPEC, _W_SPEC, _V_SPEC, _V_SPEC,
              _V_SPEC],
    out_specs=_row_spec(BLK),
    out_shape=jax.ShapeDtypeStruct((N, C), jnp.float32),
)

_tc_out = pl.pallas_call(
    _tc_out_body,
    grid=(N // BLK,),
    in_specs=[_P_SPEC, _row_spec(BLK), _row_spec(BLK), _W_SPEC, _W_SPEC,
              _V_SPEC, _V_SPEC, _V_SPEC, _W_SPEC, _V_SPEC],
    out_specs=[_row_spec(BLK), _row_spec(BLK)],
    out_shape=[jax.ShapeDtypeStruct((N, C), jnp.float32),
               jax.ShapeDtypeStruct((N, C), jnp.float32)],
)


def kernel(x, edge, edgeweight, W_in, b_in, Wrel0, brel0, Wroot0, g0, be0,
           Wrel1, brel1, Wroot1, g1, be1, W_out, b_out):
    zeros = jnp.zeros((N, C), jnp.float32)
    h = _tc_in(x, W_in.T, b_in.reshape(1, C))
    p0 = _segsum()(h, edge, edgeweight, zeros)
    x1 = _tc_comb(p0, h, Wrel0.T, Wroot0.T, brel0.reshape(1, C),
                  g0.reshape(1, C), be0.reshape(1, C))
    p1 = _segsum()(x1, edge, edgeweight, zeros)
    x2, out = _tc_out(p1, x1, h, Wrel1.T, Wroot1.T, brel1.reshape(1, C),
                      g1.reshape(1, C), be1.reshape(1, C), W_out.T,
                      b_out.reshape(1, C))
    return (x2, out)
